# baseline (speedup 1.0000x reference)
import jax
import jax.numpy as jnp
from jax.experimental import pallas as pl
from jax.experimental.pallas import tpu as pltpu


def _gate_block_kernel(xt_ref, w1_ref, b1_ref, w2t_ref, b2_ref,
                       out_ref, att_ref):
    xv = xt_ref[...]
    hw = xv.shape[0]

    ssum = jnp.sum(xv, axis=0, dtype=jnp.float32)
    smax = jnp.max(xv, axis=0)
    avg = ssum * (1.0 / hw)

    pooled = jnp.concatenate([avg, smax], axis=0)
    h = jax.lax.dot_general(pooled, w1_ref[...],
                            (((1,), (1,)), ((), ())),
                            preferred_element_type=jnp.float32)
    h = jnp.maximum(h + b1_ref[...], 0.0)
    a2 = jnp.dot(h, w2t_ref[...], preferred_element_type=jnp.float32)
    a2 = a2 + b2_ref[...]
    bt = avg.shape[0]
    att = a2[:bt] + a2[bt:]

    att_ref[...] = att
    out_ref[...] = xv * jax.nn.sigmoid(att)[None, :, :].astype(out_ref.dtype)


def _pick_bt(B, C, HW, itemsize):
    budget = 44 << 20
    for bt in (8, 4, 2, 1):
        if B % bt:
            continue
        if 4 * HW * bt * C * itemsize <= budget:
            return bt
    return 1


def kernel(x, w1, b1, w2, b2):
    B, C, H, W = x.shape
    HW = H * W
    hidden = w1.shape[0]

    xt = jnp.transpose(x.reshape(B, C, HW), (2, 0, 1))
    itemsize = jnp.dtype(x.dtype).itemsize
    bt = _pick_bt(B, C, HW, itemsize)
    nblocks = B // bt

    b1r = b1.reshape(1, hidden).astype(jnp.float32)
    w2t = jnp.transpose(w2).astype(jnp.float32)
    b2r = b2.reshape(1, C).astype(jnp.float32)

    xbytes = B * C * HW * itemsize
    cost = pl.CostEstimate(
        flops=3 * B * C * HW + 8 * B * C * hidden,
        transcendentals=B * C,
        bytes_accessed=2 * xbytes + 4 * B * C + 8 * C * hidden,
    )

    out_t, att = pl.pallas_call(
        _gate_block_kernel,
        out_shape=(
            jax.ShapeDtypeStruct((HW, B, C), x.dtype),
            jax.ShapeDtypeStruct((B, C), jnp.float32),
        ),
        grid=(nblocks,),
        in_specs=[
            pl.BlockSpec((HW, bt, C), lambda b: (0, b, 0)),
            pl.BlockSpec((hidden, C), lambda b: (0, 0)),
            pl.BlockSpec((1, hidden), lambda b: (0, 0)),
            pl.BlockSpec((hidden, C), lambda b: (0, 0)),
            pl.BlockSpec((1, C), lambda b: (0, 0)),
        ],
        out_specs=(
            pl.BlockSpec((HW, bt, C), lambda b: (0, b, 0)),
            pl.BlockSpec((bt, C), lambda b: (b, 0)),
        ),
        compiler_params=pltpu.CompilerParams(
            dimension_semantics=("parallel",),
            vmem_limit_bytes=56 << 20),
        cost_estimate=cost,
    )(xt, w1, b1r, w2t, b2r)

    out = jnp.transpose(out_t, (1, 2, 0)).reshape(B, C, H, W)
    return out, jnp.mean(jnp.abs(att))


# speedup vs baseline: 1.1143x; 1.1143x over previous
import jax
import jax.numpy as jnp
from jax.experimental import pallas as pl
from jax.experimental.pallas import tpu as pltpu


def _gate_block_kernel(xt_ref, w1_ref, b1_ref, w2t_ref, b2_ref,
                       out_ref, att_ref):
    xv = xt_ref[...]
    hw = xv.shape[0]

    ssum = jnp.sum(xv, axis=0, dtype=jnp.float32)
    smax = jnp.max(xv, axis=0)
    avg = ssum * (1.0 / hw)

    pooled = jnp.concatenate([avg, smax], axis=0)
    h = jax.lax.dot_general(pooled, w1_ref[...],
                            (((1,), (1,)), ((), ())),
                            preferred_element_type=jnp.float32)
    h = jnp.maximum(h + b1_ref[...], 0.0)
    a2 = jnp.dot(h, w2t_ref[...], preferred_element_type=jnp.float32)
    a2 = a2 + b2_ref[...]
    bt = avg.shape[0]
    att = a2[:bt] + a2[bt:]

    att_ref[...] = att
    out_ref[...] = xv * jax.nn.sigmoid(att)[None, :, :].astype(out_ref.dtype)


def _pick_bt(B, C, HW, itemsize):
    budget = 52 << 20
    for bt in (32, 16, 8, 4, 2, 1):
        if B % bt:
            continue
        if 4 * HW * bt * C * itemsize <= budget:
            return bt
    return 1


def kernel(x, w1, b1, w2, b2):
    B, C, H, W = x.shape
    HW = H * W
    hidden = w1.shape[0]

    xt = jnp.transpose(x.reshape(B, C, HW), (2, 0, 1))
    itemsize = jnp.dtype(x.dtype).itemsize
    bt = _pick_bt(B, C, HW, itemsize)
    nblocks = B // bt

    b1r = b1.reshape(1, hidden).astype(jnp.float32)
    w2t = jnp.transpose(w2).astype(jnp.float32)
    b2r = b2.reshape(1, C).astype(jnp.float32)

    xbytes = B * C * HW * itemsize
    cost = pl.CostEstimate(
        flops=3 * B * C * HW + 8 * B * C * hidden,
        transcendentals=B * C,
        bytes_accessed=2 * xbytes + 4 * B * C + 8 * C * hidden,
    )

    out_t, att = pl.pallas_call(
        _gate_block_kernel,
        out_shape=(
            jax.ShapeDtypeStruct((HW, B, C), x.dtype),
            jax.ShapeDtypeStruct((B, C), jnp.float32),
        ),
        grid=(nblocks,),
        in_specs=[
            pl.BlockSpec((HW, bt, C), lambda b: (0, b, 0)),
            pl.BlockSpec((hidden, C), lambda b: (0, 0)),
            pl.BlockSpec((1, hidden), lambda b: (0, 0)),
            pl.BlockSpec((hidden, C), lambda b: (0, 0)),
            pl.BlockSpec((1, C), lambda b: (0, 0)),
        ],
        out_specs=(
            pl.BlockSpec((HW, bt, C), lambda b: (0, b, 0)),
            pl.BlockSpec((bt, C), lambda b: (b, 0)),
        ),
        compiler_params=pltpu.CompilerParams(
            dimension_semantics=("parallel",),
            vmem_limit_bytes=56 << 20),
        cost_estimate=cost,
    )(xt, w1, b1r, w2t, b2r)

    out = jnp.transpose(out_t, (1, 2, 0)).reshape(B, C, H, W)
    return out, jnp.mean(jnp.abs(att))


# speedup vs baseline: 1.1170x; 1.0025x over previous
import jax
import jax.numpy as jnp
from jax.experimental import pallas as pl
from jax.experimental.pallas import tpu as pltpu


def _gate_block_kernel(xt_ref, w1_ref, b1_ref, w2t_ref, b2_ref,
                       out_ref, pabs_ref):
    xv = xt_ref[...]
    hw = xv.shape[0]

    ssum = jnp.sum(xv, axis=0, dtype=jnp.float32)
    smax = jnp.max(xv, axis=0)
    avg = ssum * (1.0 / hw)

    pooled = jnp.concatenate([avg, smax], axis=0)
    h = jax.lax.dot_general(pooled, w1_ref[...],
                            (((1,), (1,)), ((), ())),
                            preferred_element_type=jnp.float32)
    h = jnp.maximum(h + b1_ref[...], 0.0)
    a2 = jnp.dot(h, w2t_ref[...], preferred_element_type=jnp.float32)
    a2 = a2 + b2_ref[...]
    bt = avg.shape[0]
    att = a2[:bt] + a2[bt:]

    pabs_ref[...] = jnp.sum(jnp.abs(att), axis=0, keepdims=True)[None]
    out_ref[...] = xv * jax.nn.sigmoid(att)[None, :, :].astype(out_ref.dtype)


def _pick_bt(B, C, HW, itemsize):
    budget = 52 << 20
    for bt in (32, 16, 8, 4, 2, 1):
        if B % bt:
            continue
        if 4 * HW * bt * C * itemsize <= budget:
            return bt
    return 1


def kernel(x, w1, b1, w2, b2):
    B, C, H, W = x.shape
    HW = H * W
    hidden = w1.shape[0]

    xt = jnp.transpose(x.reshape(B, C, HW), (2, 0, 1))
    itemsize = jnp.dtype(x.dtype).itemsize
    bt = _pick_bt(B, C, HW, itemsize)
    nblocks = B // bt

    b1r = b1.reshape(1, hidden).astype(jnp.float32)
    w2t = jnp.transpose(w2).astype(jnp.float32)
    b2r = b2.reshape(1, C).astype(jnp.float32)

    xbytes = B * C * HW * itemsize
    cost = pl.CostEstimate(
        flops=3 * B * C * HW + 8 * B * C * hidden,
        transcendentals=B * C,
        bytes_accessed=2 * xbytes + 4 * B * C + 8 * C * hidden,
    )

    out_t, pabs = pl.pallas_call(
        _gate_block_kernel,
        out_shape=(
            jax.ShapeDtypeStruct((HW, B, C), x.dtype),
            jax.ShapeDtypeStruct((nblocks, 1, C), jnp.float32),
        ),
        grid=(nblocks,),
        in_specs=[
            pl.BlockSpec((HW, bt, C), lambda b: (0, b, 0)),
            pl.BlockSpec((hidden, C), lambda b: (0, 0)),
            pl.BlockSpec((1, hidden), lambda b: (0, 0)),
            pl.BlockSpec((hidden, C), lambda b: (0, 0)),
            pl.BlockSpec((1, C), lambda b: (0, 0)),
        ],
        out_specs=(
            pl.BlockSpec((HW, bt, C), lambda b: (0, b, 0)),
            pl.BlockSpec((1, 1, C), lambda b: (b, 0, 0)),
        ),
        compiler_params=pltpu.CompilerParams(
            dimension_semantics=("parallel",),
            vmem_limit_bytes=56 << 20),
        cost_estimate=cost,
    )(xt, w1, b1r, w2t, b2r)

    out = jnp.transpose(out_t, (1, 2, 0)).reshape(B, C, H, W)
    return out, jnp.sum(pabs) * (1.0 / (B * C))
